# baseline (speedup 1.0000x reference)
import functools

import jax
import jax.numpy as jnp
from jax import lax
from jax.experimental import pallas as pl
from jax.experimental.pallas import tpu as pltpu
from jax.experimental.pallas import tpu_sc as plsc


def _pooled_sum_sc(input_ids, embed_table):
    B, S = input_ids.shape
    V, D = embed_table.shape
    NW = 32
    b_per_w = B // NW
    L = 16
    nchunk = D // L
    C0 = 128
    C1 = S - C0

    mesh = plsc.VectorSubcoreMesh(core_axis_name="c", subcore_axis_name="s")

    @functools.partial(
        pl.kernel,
        out_type=jax.ShapeDtypeStruct((B, D), jnp.float32),
        mesh=mesh,
        scratch_types=[
            pltpu.VMEM((S,), jnp.int32),
            pltpu.VMEM((S, D), jnp.float32),
            pltpu.VMEM((D,), jnp.float32),
            pltpu.SemaphoreType.DMA,
        ],
        compiler_params=pltpu.CompilerParams(use_tc_tiling_on_sc=False),
    )
    def pooled(ids_hbm, table_hbm, out_hbm, idx_v, rows_v, acc_v, sem):
        wid = lax.axis_index("s") * 2 + lax.axis_index("c")
        base = wid * b_per_w

        def row_body(i, carry):
            row = base + i
            pltpu.sync_copy(ids_hbm.at[row], idx_v)
            cp0 = pltpu.async_copy(
                table_hbm.at[idx_v.at[pl.ds(0, C0)]],
                rows_v.at[pl.ds(0, C0)], sem)
            cp1 = pltpu.async_copy(
                table_hbm.at[idx_v.at[pl.ds(C0, C1)]],
                rows_v.at[pl.ds(C0, C1)], sem)
            cp0.wait()
            cp1.wait()

            U = 8
            def red(t, accs):
                s0 = t * U
                out = list(accs)
                for u in range(U):
                    for k in range(nchunk):
                        out[k] = out[k] + rows_v[s0 + u, pl.ds(k * L, L)]
                return tuple(out)

            zeros = tuple(jnp.zeros((L,), jnp.float32) for _ in range(nchunk))
            accs = lax.fori_loop(0, S // U, red, zeros)
            for k in range(nchunk):
                acc_v[pl.ds(k * L, L)] = accs[k]
            pltpu.sync_copy(acc_v, out_hbm.at[row])
            return carry

        lax.fori_loop(0, b_per_w, row_body, 0)

    return pooled(input_ids, embed_table)


def _linear_tc(pooled, Ws, b2d):
    B, D = pooled.shape
    C = Ws.shape[1]
    BM = 512

    def mm(x_ref, w_ref, b_ref, o_ref):
        o_ref[...] = (
            jnp.dot(x_ref[...], w_ref[...], preferred_element_type=jnp.float32)
            + b_ref[...]
        )

    return pl.pallas_call(
        mm,
        grid=(B // BM,),
        in_specs=[
            pl.BlockSpec((BM, D), lambda i: (i, 0)),
            pl.BlockSpec((D, C), lambda i: (0, 0)),
            pl.BlockSpec((1, C), lambda i: (0, 0)),
        ],
        out_specs=pl.BlockSpec((BM, C), lambda i: (i, 0)),
        out_shape=jax.ShapeDtypeStruct((B, C), jnp.float32),
    )(pooled, Ws, b2d)


@jax.jit
def kernel(input_ids, embed_table, W, b):
    S = input_ids.shape[1]
    pooled = _pooled_sum_sc(input_ids.astype(jnp.int32), embed_table)
    Ws = W * (1.0 / S)
    b2d = b.reshape(1, -1)
    return _linear_tc(pooled, Ws, b2d)


# speedup vs baseline: 1.2183x; 1.2183x over previous
import functools

import jax
import jax.numpy as jnp
from jax import lax
from jax.experimental import pallas as pl
from jax.experimental.pallas import tpu as pltpu
from jax.experimental.pallas import tpu_sc as plsc


def _pooled_sum_sc(input_ids, embed_table):
    B, S = input_ids.shape
    V, D = embed_table.shape
    NW = 32
    b_per_w = B // NW
    L = 16
    nchunk = D // L
    C0 = 128
    C1 = S - C0
    U = 8

    mesh = plsc.VectorSubcoreMesh(core_axis_name="c", subcore_axis_name="s")

    @functools.partial(
        pl.kernel,
        out_type=jax.ShapeDtypeStruct((B, D), jnp.float32),
        mesh=mesh,
        scratch_types=[
            pltpu.VMEM((b_per_w, S), jnp.int32),
            pltpu.VMEM((2, S, D), jnp.float32),
            pltpu.VMEM((b_per_w, D), jnp.float32),
            pltpu.SemaphoreType.DMA((2,)),
        ],
        compiler_params=pltpu.CompilerParams(use_tc_tiling_on_sc=False),
    )
    def pooled(ids_hbm, table_hbm, out_hbm, idx_all, rows_v, out_all, sems):
        wid = lax.axis_index("s") * 2 + lax.axis_index("c")
        base = wid * b_per_w

        pltpu.sync_copy(ids_hbm.at[pl.ds(base, b_per_w)], idx_all)

        def fire(buf, row):
            @pl.when(row < b_per_w)
            def _():
                pltpu.async_copy(
                    table_hbm.at[idx_all.at[row, pl.ds(0, C0)]],
                    rows_v.at[buf, pl.ds(0, C0)], sems.at[buf])
                pltpu.async_copy(
                    table_hbm.at[idx_all.at[row, pl.ds(C0, C1)]],
                    rows_v.at[buf, pl.ds(C0, C1)], sems.at[buf])

        def drain(buf):
            pltpu.make_async_copy(
                table_hbm.at[pl.ds(0, S)], rows_v.at[buf], sems.at[buf]
            ).wait()

        fire(0, jnp.int32(0))
        fire(1, jnp.int32(1))

        def pair_body(p, carry):
            for k in range(2):
                row = p * 2 + k
                drain(k)

                def red(t, accs):
                    s0 = t * U
                    out = list(accs)
                    for u in range(U):
                        for c in range(nchunk):
                            out[c] = out[c] + rows_v[k, s0 + u, pl.ds(c * L, L)]
                    return tuple(out)

                zeros = tuple(
                    jnp.zeros((L,), jnp.float32) for _ in range(nchunk))
                accs = lax.fori_loop(0, S // U, red, zeros)
                fire(k, row + 2)
                for c in range(nchunk):
                    out_all[row, pl.ds(c * L, L)] = accs[c]
            return carry

        lax.fori_loop(0, b_per_w // 2, pair_body, 0)
        pltpu.sync_copy(out_all, out_hbm.at[pl.ds(base, b_per_w)])

    return pooled(input_ids, embed_table)


def _linear_tc(pooled, Ws, b2d):
    B, D = pooled.shape
    C = Ws.shape[1]
    BM = 512

    def mm(x_ref, w_ref, b_ref, o_ref):
        o_ref[...] = (
            jnp.dot(x_ref[...], w_ref[...], preferred_element_type=jnp.float32)
            + b_ref[...]
        )

    return pl.pallas_call(
        mm,
        grid=(B // BM,),
        in_specs=[
            pl.BlockSpec((BM, D), lambda i: (i, 0)),
            pl.BlockSpec((D, C), lambda i: (0, 0)),
            pl.BlockSpec((1, C), lambda i: (0, 0)),
        ],
        out_specs=pl.BlockSpec((BM, C), lambda i: (i, 0)),
        out_shape=jax.ShapeDtypeStruct((B, C), jnp.float32),
    )(pooled, Ws, b2d)


@jax.jit
def kernel(input_ids, embed_table, W, b):
    S = input_ids.shape[1]
    pooled = _pooled_sum_sc(input_ids.astype(jnp.int32), embed_table)
    Ws = W * (1.0 / S)
    b2d = b.reshape(1, -1)
    return _linear_tc(pooled, Ws, b2d)
